# Initial kernel scaffold; baseline (speedup 1.0000x reference)
import jax
import jax.numpy as jnp
from jax.experimental import pallas as pl


def kernel(x, edge_index, gn0_w, gn0_b, gn0_ms, W1, b1, a1, gn1_w, gn1_b, gn1_ms, W2, b2, a2, gn2_w, gn2_b, gn2_ms, W3, b3, gn3_w, gn3_b, gn3_ms, Wih, Whh, bih, bhh):
    raise NotImplementedError("write your pallas kernel here")


# speedup vs baseline: 7.5827x; 7.5827x over previous
import functools

import jax
import jax.numpy as jnp
from jax import lax
from jax.experimental import pallas as pl
from jax.experimental.pallas import tpu as pltpu
from jax.experimental.pallas import tpu_sc as plsc

N = 50000
E = 800000
H = 128
D_IN = 4

NC = 2
NS = 16
NT = NC * NS
CW = 128
CH = 196
EP = NT * CH * CW
NQ = 8
QW = H // NQ
NP = 51200
RPT = NP // NS
ZR = 400

BN = 2048
NB = NP // BN
FLT_N = float(N)

@functools.cache
def _sc_mesh():
    return plsc.VectorSubcoreMesh(core_axis_name="c", subcore_axis_name="s",
                                  num_cores=NC, num_subcores=NS)


def _deg_body(col_hbm, ones_hbm, zeros_hbm, cnt_hbm, idx_v, ones_v, zero_v,
              acc_s):
    c = lax.axis_index("c")
    s = lax.axis_index("s")
    tid = c * NS + s
    pltpu.sync_copy(col_hbm.at[tid], idx_v)
    pltpu.sync_copy(ones_hbm, ones_v)
    pltpu.sync_copy(zeros_hbm, zero_v)

    def zero_body(j, _):
        pltpu.sync_copy(zero_v, acc_s.at[pl.ds(s * RPT + j * ZR, ZR)])
        return 0
    lax.fori_loop(0, RPT // ZR, zero_body, 0)
    plsc.subcore_barrier()

    def chunk_body(i, _):
        pltpu.sync_copy(ones_v, acc_s.at[idx_v.at[i]], add=True)
        return 0
    lax.fori_loop(0, CH, chunk_body, 0)
    plsc.subcore_barrier()
    pltpu.sync_copy(acc_s.at[pl.ds(s * RPT, RPT)],
                    cnt_hbm.at[pl.ds(c * NP + s * RPT, RPT)])


@functools.cache
def _deg_kernel():
    return pl.kernel(
        _deg_body,
        out_type=jax.ShapeDtypeStruct((NC * NP, 16), jnp.float32),
        mesh=_sc_mesh(),
        scratch_types=[
            pltpu.VMEM((CH, CW), jnp.int32),
            pltpu.VMEM((CW, 16), jnp.float32),
            pltpu.VMEM((ZR, 16), jnp.float32),
            pltpu.VMEM_SHARED((NP, 16), jnp.float32),
        ],
        compiler_params=pltpu.CompilerParams(use_tc_tiling_on_sc=False),
    )


def _deg_call(*args):
    return _deg_kernel()(*args)


def _conv_body(rowo_hbm, col_hbm, g_hbm, zeros_hbm, out_hbm, idxr_v, idxc_v,
               rows_v, zero_v, acc_s, sem):
    c = lax.axis_index("c")
    s = lax.axis_index("s")
    tid = c * NS + s
    pltpu.sync_copy(col_hbm.at[tid], idxc_v)
    pltpu.sync_copy(zeros_hbm, zero_v)

    for p in range(NQ // NC):
        q = c * (NQ // NC) + p
        pltpu.sync_copy(rowo_hbm.at[q * NT + tid], idxr_v)

        def zero_body(j, _):
            pltpu.sync_copy(zero_v, acc_s.at[pl.ds(s * RPT + j * ZR, ZR)])
            return 0
        lax.fori_loop(0, RPT // ZR, zero_body, 0)
        plsc.subcore_barrier()

        def chunk_body(i, _):
            pltpu.async_copy(g_hbm.at[idxr_v.at[i]], rows_v, sem).wait()
            pltpu.sync_copy(rows_v, acc_s.at[idxc_v.at[i]], add=True)
            return 0
        lax.fori_loop(0, CH, chunk_body, 0)
        plsc.subcore_barrier()
        pltpu.sync_copy(acc_s.at[pl.ds(s * RPT, RPT)],
                        out_hbm.at[pl.ds(q * NP + s * RPT, RPT)])
        plsc.subcore_barrier()


@functools.cache
def _conv_kernel():
    return pl.kernel(
        _conv_body,
        out_type=jax.ShapeDtypeStruct((NQ * NP, QW), jnp.float32),
        mesh=_sc_mesh(),
        scratch_types=[
            pltpu.VMEM((CH, CW), jnp.int32),
            pltpu.VMEM((CH, CW), jnp.int32),
            pltpu.VMEM((CW, QW), jnp.float32),
            pltpu.VMEM((ZR, QW), jnp.float32),
            pltpu.VMEM_SHARED((NP, QW), jnp.float32),
            pltpu.SemaphoreType.DMA,
        ],
        compiler_params=pltpu.CompilerParams(use_tc_tiling_on_sc=False),
    )


def _conv_call(*args):
    return _conv_kernel()(*args)


def _rowmask(pid):
    grow = pid * BN + lax.broadcasted_iota(jnp.int32, (BN, 1), 0)
    return grow < N


def _t0a_body(x_ref, sums_ref):
    pid = pl.program_id(0)
    xb = x_ref[...]
    s1 = jnp.sum(xb, axis=0, keepdims=True)
    s2 = jnp.sum(xb * xb, axis=0, keepdims=True)
    full = jnp.concatenate(
        [jnp.pad(s1, ((0, 0), (0, H - D_IN))),
         jnp.pad(s2, ((0, 0), (0, H - D_IN))),
         jnp.zeros((6, H), jnp.float32)], axis=0)

    @pl.when(pid == 0)
    def _():
        sums_ref[...] = jnp.zeros_like(sums_ref)
    sums_ref[...] += full


def _t0b_body(x_ref, cnt_ref, sums_ref, w1_ref, gw_ref, gb_ref, gms_ref,
              g_ref, dinv_ref):
    pid = pl.program_id(0)
    mask = _rowmask(pid)
    ms = gms_ref[...]
    m = sums_ref[0:1, 0:D_IN] / FLT_N
    msq = sums_ref[1:2, 0:D_IN] / FLT_N
    var = msq - m * m * ms * (2.0 - ms)
    rstd = lax.rsqrt(var + 1e-5)
    xn = (x_ref[...] - ms * m) * rstd * gw_ref[...] + gb_ref[...]
    deg = cnt_ref[0, :, 0:1] + cnt_ref[1, :, 0:1] + 1.0
    dinv = jnp.where(mask, lax.rsqrt(deg), 0.0)
    hw = jnp.dot(xn, w1_ref[...], preferred_element_type=jnp.float32)
    g = hw * dinv
    for q in range(NQ):
        g_ref[q, :, :] = g[:, q * QW:(q + 1) * QW]
    dinv_ref[...] = jnp.broadcast_to(dinv, (BN, 8))


def _tr_body(g_ref, acc_ref, dinv_ref, b_ref, a_ref, t_ref, sums_ref, *,
             prelu):
    pid = pl.program_id(0)
    mask = _rowmask(pid)
    gb = jnp.concatenate([g_ref[q] for q in range(NQ)], axis=1)
    ab = jnp.concatenate([acc_ref[q] for q in range(NQ)], axis=1)
    dv = dinv_ref[:, 0:1]
    t = dv * (gb + ab) + b_ref[...]
    if prelu:
        t = jnp.where(t >= 0.0, t, a_ref[...] * t)
    t = jnp.where(mask, t, 0.0)
    t_ref[...] = t
    s1 = jnp.sum(t, axis=0, keepdims=True)
    s2 = jnp.sum(t * t, axis=0, keepdims=True)
    full = jnp.concatenate([s1, s2, jnp.zeros((6, H), jnp.float32)], axis=0)

    @pl.when(pid == 0)
    def _():
        sums_ref[...] = jnp.zeros_like(sums_ref)
    sums_ref[...] += full


def _ta_body(t_ref, sums_ref, w_ref, dinv_ref, gw_ref, gb_ref, gms_ref,
             g_ref):
    ms = gms_ref[...]
    m = sums_ref[0:1, :] / FLT_N
    msq = sums_ref[1:2, :] / FLT_N
    var = msq - m * m * ms * (2.0 - ms)
    rstd = lax.rsqrt(var + 1e-5)
    tn = (t_ref[...] - ms * m) * rstd * gw_ref[...] + gb_ref[...]
    h = jnp.dot(tn, w_ref[...], preferred_element_type=jnp.float32)
    h = h * dinv_ref[:, 0:1]
    for q in range(NQ):
        g_ref[q, :, :] = h[:, q * QW:(q + 1) * QW]


_SB = 2048
_NSB = NP // _SB


_PH = lax.Precision.HIGHEST


def _s2s_body(t_ref, gw_ref, gb_ref, gms_ref, wih_ref, whh_ref,
              bih_ref, bhh_ref, out_ref):
    ms = gms_ref[...]
    zrow = jnp.zeros((1, H), jnp.float32)
    npad = float(NP - N)

    def blk(i):
        return t_ref[pl.ds(i * _SB, _SB), :]

    def kadd(acc, comp, p):
        y = p - comp
        tn = acc + y
        return tn, (tn - acc) - y

    def sum_pass(i, acc):
        return acc + jnp.sum(blk(i), axis=0, keepdims=True)
    s0 = lax.fori_loop(0, _NSB, sum_pass, zrow)
    m0 = s0 / FLT_N

    def r0_pass(i, c):
        acc, comp = c
        return kadd(acc, comp, jnp.sum(blk(i) - m0, axis=0, keepdims=True))
    r0, _ = lax.fori_loop(0, _NSB, r0_pass, (zrow, zrow))
    r0 = r0 + npad * m0
    mt = ms * (m0 + r0 / FLT_N)

    def var_pass(i, acc):
        cen = blk(i) - mt
        return acc + jnp.sum(cen * cen, axis=0, keepdims=True)
    sq = lax.fori_loop(0, _NSB, var_pass, zrow) - npad * (mt * mt)
    rstd = lax.rsqrt(sq / FLT_N + 1e-5)
    k = rstd * gw_ref[...]
    gb = gb_ref[...]

    h = jnp.zeros((1, H), jnp.float32)
    cvec = jnp.zeros((1, H), jnp.float32)
    q_star = jnp.zeros((1, 2 * H), jnp.float32)
    for _ in range(3):
        gates = (lax.dot_general(q_star, wih_ref[...],
                                 (((1,), (1,)), ((), ())),
                                 precision=_PH,
                                 preferred_element_type=jnp.float32)
                 + bih_ref[...]
                 + lax.dot_general(h, whh_ref[...],
                                   (((1,), (1,)), ((), ())),
                                   precision=_PH,
                                   preferred_element_type=jnp.float32)
                 + bhh_ref[...])
        ig = jax.nn.sigmoid(gates[:, 0:H])
        fg = jax.nn.sigmoid(gates[:, H:2 * H])
        gg = jnp.tanh(gates[:, 2 * H:3 * H])
        og = jax.nn.sigmoid(gates[:, 3 * H:4 * H])
        cvec = fg * cvec + ig * gg
        h = og * jnp.tanh(cvec)
        kh = k * h

        def e_of(i):
            e = lax.dot_general(blk(i) - mt, kh, (((1,), (1,)), ((), ())),
                                precision=_PH,
                                preferred_element_type=jnp.float32)
            grow = i * _SB + lax.broadcasted_iota(jnp.int32, (_SB, 1), 0)
            return jnp.where(grow < N, e, -1e30)

        def emax_pass(i, acc):
            return jnp.maximum(acc, jnp.max(e_of(i), axis=0, keepdims=True))
        emax = lax.fori_loop(0, _NSB, emax_pass,
                             jnp.full((1, 1), -1e30, jnp.float32))

        def sr_pass(i, c):
            ssum, racc, comp = c
            a = jnp.exp(e_of(i) - emax)
            ssum = ssum + jnp.sum(a, axis=0, keepdims=True)
            p = lax.dot_general(a, blk(i) - mt, (((0,), (0,)), ((), ())),
                                precision=_PH,
                                preferred_element_type=jnp.float32)
            racc, comp = kadd(racc, comp, p)
            return (ssum, racc, comp)
        ssum, racc, _ = lax.fori_loop(
            0, _NSB, sr_pass, (jnp.zeros((1, 1), jnp.float32), zrow, zrow))
        r = (racc / ssum) * k + gb
        q_star = jnp.concatenate([h, r], axis=1)
    out_ref[...] = q_star


def _blk(shape, imap):
    return pl.BlockSpec(shape, imap)


_t0a = pl.pallas_call(
    _t0a_body,
    grid=(NB,),
    in_specs=[_blk((BN, D_IN), lambda i: (i, 0))],
    out_specs=_blk((8, H), lambda i: (0, 0)),
    out_shape=jax.ShapeDtypeStruct((8, H), jnp.float32),
)

_t0b = pl.pallas_call(
    _t0b_body,
    grid=(NB,),
    in_specs=[
        _blk((BN, D_IN), lambda i: (i, 0)),
        _blk((NC, BN, 16), lambda i: (0, i, 0)),
        _blk((8, H), lambda i: (0, 0)),
        _blk((D_IN, H), lambda i: (0, 0)),
        _blk((1, D_IN), lambda i: (0, 0)),
        _blk((1, D_IN), lambda i: (0, 0)),
        _blk((1, D_IN), lambda i: (0, 0)),
    ],
    out_specs=[
        _blk((NQ, BN, QW), lambda i: (0, i, 0)),
        _blk((BN, 8), lambda i: (i, 0)),
    ],
    out_shape=[
        jax.ShapeDtypeStruct((NQ, NP, QW), jnp.float32),
        jax.ShapeDtypeStruct((NP, 8), jnp.float32),
    ],
)


def _make_tr(prelu):
    return pl.pallas_call(
        functools.partial(_tr_body, prelu=prelu),
        grid=(NB,),
        in_specs=[
            _blk((NQ, BN, QW), lambda i: (0, i, 0)),
            _blk((NQ, BN, QW), lambda i: (0, i, 0)),
            _blk((BN, 8), lambda i: (i, 0)),
            _blk((1, H), lambda i: (0, 0)),
            _blk((1, H), lambda i: (0, 0)),
        ],
        out_specs=[
            _blk((BN, H), lambda i: (i, 0)),
            _blk((8, H), lambda i: (0, 0)),
        ],
        out_shape=[
            jax.ShapeDtypeStruct((NP, H), jnp.float32),
            jax.ShapeDtypeStruct((8, H), jnp.float32),
        ],
    )


_tr_prelu = _make_tr(True)
_tr_plain = _make_tr(False)

_ta = pl.pallas_call(
    _ta_body,
    grid=(NB,),
    in_specs=[
        _blk((BN, H), lambda i: (i, 0)),
        _blk((8, H), lambda i: (0, 0)),
        _blk((H, H), lambda i: (0, 0)),
        _blk((BN, 8), lambda i: (i, 0)),
        _blk((1, H), lambda i: (0, 0)),
        _blk((1, H), lambda i: (0, 0)),
        _blk((1, H), lambda i: (0, 0)),
    ],
    out_specs=_blk((NQ, BN, QW), lambda i: (0, i, 0)),
    out_shape=jax.ShapeDtypeStruct((NQ, NP, QW), jnp.float32),
)

_s2s = pl.pallas_call(
    _s2s_body,
    in_specs=[
        _blk((NP, H), lambda: (0, 0)),
        _blk((1, H), lambda: (0, 0)),
        _blk((1, H), lambda: (0, 0)),
        _blk((1, H), lambda: (0, 0)),
        _blk((4 * H, 2 * H), lambda: (0, 0)),
        _blk((4 * H, H), lambda: (0, 0)),
        _blk((1, 4 * H), lambda: (0, 0)),
        _blk((1, 4 * H), lambda: (0, 0)),
    ],
    out_specs=_blk((1, 2 * H), lambda: (0, 0)),
    out_shape=jax.ShapeDtypeStruct((1, 2 * H), jnp.float32),
)


def kernel(x, edge_index, gn0_w, gn0_b, gn0_ms, W1, b1, a1, gn1_w, gn1_b,
           gn1_ms, W2, b2, a2, gn2_w, gn2_b, gn2_ms, W3, b3, gn3_w, gn3_b,
           gn3_ms, Wih, Whh, bih, bhh):
    f32 = jnp.float32
    xp = jnp.pad(x, ((0, NP - N), (0, 0)))
    pad = jnp.full((EP - E,), N, jnp.int32)
    rowp = jnp.concatenate([edge_index[0], pad])
    colp = jnp.concatenate([edge_index[1], pad])
    col_r = colp.reshape(NT, CH, CW)
    rowo = (rowp[None, :]
            + (jnp.arange(NQ, dtype=jnp.int32) * NP)[:, None])
    rowo = rowo.reshape(NQ * NT, CH, CW)
    ones16 = jnp.ones((CW, 16), f32)
    zeros16 = jnp.zeros((ZR, 16), f32)
    zerosq = jnp.zeros((ZR, QW), f32)

    r4 = lambda v: v.reshape(1, -1).astype(f32)

    cnt = _deg_call(col_r, ones16, zeros16)
    sums0 = _t0a(xp)
    g1, dinv = _t0b(xp, cnt.reshape(NC, NP, 16), sums0, W1, r4(gn0_w),
                    r4(gn0_b), r4(gn0_ms))

    acc1 = _conv_call(rowo, col_r, g1.reshape(NQ * NP, QW), zerosq)
    t1, s1 = _tr_prelu(g1, acc1.reshape(NQ, NP, QW), dinv, r4(b1), r4(a1))
    g2 = _ta(t1, s1, W2, dinv, r4(gn1_w), r4(gn1_b), r4(gn1_ms))

    acc2 = _conv_call(rowo, col_r, g2.reshape(NQ * NP, QW), zerosq)
    t2, s2 = _tr_prelu(g2, acc2.reshape(NQ, NP, QW), dinv, r4(b2), r4(a2))
    g3 = _ta(t2, s2, W3, dinv, r4(gn2_w), r4(gn2_b), r4(gn2_ms))

    acc3 = _conv_call(rowo, col_r, g3.reshape(NQ * NP, QW), zerosq)
    t3, _ = _tr_plain(g3, acc3.reshape(NQ, NP, QW), dinv, r4(b3), r4(b3))

    return _s2s(t3, r4(gn3_w), r4(gn3_b), r4(gn3_ms), Wih, Whh,
                r4(bih), r4(bhh))
